# Initial kernel scaffold; baseline (speedup 1.0000x reference)
import jax
import jax.numpy as jnp
from jax.experimental import pallas as pl


def kernel(position_ids, weight):
    raise NotImplementedError("write your pallas kernel here")


# speedup vs baseline: 2.1861x; 2.1861x over previous
import functools

import jax
import jax.numpy as jnp
from jax import lax
from jax.experimental import pallas as pl
from jax.experimental.pallas import tpu as pltpu
from jax.experimental.pallas import tpu_sc as plsc

B = 4 * 8192
D = 1024
NC, NS = 2, 16
NW = NC * NS
BPW = B // NW
C = 64


def _make_emb():
  mesh = plsc.VectorSubcoreMesh(core_axis_name="c", subcore_axis_name="s")

  @functools.partial(
      pl.kernel,
      mesh=mesh,
      out_type=jax.ShapeDtypeStruct((B, D), jnp.float32),
      scratch_types=[
          pltpu.VMEM((BPW,), jnp.int32),
          pltpu.VMEM((C, D), jnp.float32),
          pltpu.SemaphoreType.DMA,
      ],
  )
  def emb(table_hbm, idx_hbm, out_hbm, idx_v, rows_v, sem):
    wid = lax.axis_index("s") * NC + lax.axis_index("c")
    base = wid * BPW
    pltpu.sync_copy(idx_hbm.at[pl.ds(base, BPW)], idx_v)

    def body(ci, _):
      off = ci * C
      pltpu.async_copy(
          table_hbm.at[idx_v.at[pl.ds(off, C)]], rows_v, sem
      ).wait()
      pltpu.sync_copy(rows_v, out_hbm.at[pl.ds(base + off, C)])
      return ()

    lax.fori_loop(0, BPW // C, body, ())

  return emb


_emb = _make_emb()


def kernel(position_ids, weight):
  idx_flat = position_ids.reshape(-1).astype(jnp.int32)
  out = _emb(weight, idx_flat)
  return out.reshape(position_ids.shape + (D,))


# speedup vs baseline: 2.3880x; 1.0923x over previous
import functools

import jax
import jax.numpy as jnp
from jax import lax
from jax.experimental import pallas as pl
from jax.experimental.pallas import tpu as pltpu
from jax.experimental.pallas import tpu_sc as plsc

B = 4 * 8192
D = 1024
NC, NS = 2, 16
NW = NC * NS
BPW = B // NW
C = 32
NB = 2
NCH = BPW // C


def _make_emb():
  mesh = plsc.VectorSubcoreMesh(core_axis_name="c", subcore_axis_name="s")

  @functools.partial(
      pl.kernel,
      mesh=mesh,
      out_type=jax.ShapeDtypeStruct((B, D), jnp.float32),
      scratch_types=[
          pltpu.VMEM((BPW,), jnp.int32),
          pltpu.VMEM((NB, C, D), jnp.float32),
          pltpu.SemaphoreType.DMA,
          pltpu.SemaphoreType.DMA,
      ],
  )
  def emb(table_hbm, idx_hbm, out_hbm, idx_v, rows_v, gsem, ssem):
    wid = lax.axis_index("s") * NC + lax.axis_index("c")
    base = wid * BPW
    pltpu.sync_copy(idx_hbm.at[pl.ds(base, BPW)], idx_v)

    def g_desc(ci, b):
      return pltpu.make_async_copy(
          table_hbm.at[idx_v.at[pl.ds(ci * C, C)]], rows_v.at[b], gsem)

    def s_desc(ci, b):
      return pltpu.make_async_copy(
          rows_v.at[b], out_hbm.at[pl.ds(base + ci * C, C)], ssem)

    for b in range(NB):
      g_desc(b, b).start()

    @pl.loop(0, NCH, step=NB)
    def _(i):
      for b in range(NB):
        ci = i + b
        g_desc(ci, b).wait()
        s_desc(ci, b).start()
        s_desc(ci, b).wait()
        nxt = ci + NB

        @pl.when(nxt < NCH)
        def _():
          g_desc(nxt, b).start()

  return emb


_emb = _make_emb()


def kernel(position_ids, weight):
  idx_flat = position_ids.reshape(-1).astype(jnp.int32)
  out = _emb(weight, idx_flat)
  return out.reshape(position_ids.shape + (D,))
